# baseline (speedup 1.0000x reference)
import functools

import jax
import jax.numpy as jnp
from jax import lax
from jax.experimental import pallas as pl
from jax.experimental.pallas import tpu as pltpu
from jax.experimental.pallas import tpu_sc as plsc

NC = 2
NS = 16
NW = NC * NS

B, C, H, W = 4, 96, 224, 224
NPLANES = B * C
PPW = NPLANES // NW
LANE = 16


def _shuffle_body(x3, srcs, out, idx_v, buf0, buf1, gsem, ssem):
    wid = lax.axis_index("s") * NC + lax.axis_index("c")
    base = wid * PPW
    pltpu.sync_copy(srcs.at[wid], idx_v)
    ids = idx_v[...]

    def src_scalar(j):
        return ids[j]

    copies = [
        pltpu.async_copy(x3.at[src_scalar(j)], out.at[base + j], gsem)
        for j in range(PPW)
    ]
    for c in copies:
        c.wait()


@jax.jit
def _shuffle(x3, srcs):
    run = pl.kernel(
        _shuffle_body,
        out_type=jax.ShapeDtypeStruct((NPLANES, H, W), jnp.float32),
        mesh=plsc.VectorSubcoreMesh(core_axis_name="c", subcore_axis_name="s"),
        scratch_types=[
            pltpu.VMEM((LANE,), jnp.int32),
            pltpu.VMEM((H, W), jnp.float32),
            pltpu.VMEM((H, W), jnp.float32),
            pltpu.SemaphoreType.DMA,
            pltpu.SemaphoreType.DMA,
        ],
    )
    return run(x3, srcs)


def kernel(x, forward_shuffle_idx):
    src_plane = (jnp.arange(B, dtype=jnp.int32)[:, None] * C
                 + forward_shuffle_idx[None, :]).reshape(NW, PPW)
    srcs = jnp.pad(src_plane, ((0, 0), (0, LANE - PPW)))
    out = _shuffle(x.reshape(NPLANES, H, W), srcs)
    return (out.reshape(B, C, H, W), 0)


# speedup vs baseline: 32.6355x; 32.6355x over previous
import jax
import jax.numpy as jnp
from jax import lax
from jax.experimental import pallas as pl
from jax.experimental.pallas import tpu as pltpu
from jax.experimental.pallas import tpu_sc as plsc

NC = 2
NS = 16
NW = NC * NS

B, C, H, W = 4, 96, 224, 224
SPLIT = 2
CH = H // SPLIT
NCHUNK = B * C * SPLIT
CPW = NCHUNK // NW
LANE = 16
IDROWS = 2
NBUF = 4


def _shuffle_body(x3, srcs, out, idx_v, buf0, buf1, buf2, buf3, gsem, ssem):
    wid = lax.axis_index("s") * NC + lax.axis_index("c")
    base = wid * CPW
    pltpu.sync_copy(srcs.at[wid], idx_v)
    ids0 = idx_v[0, :]
    ids1 = idx_v[1, :]

    def src_scalar(c):
        return ids0[c] if c < LANE else ids1[c - LANE]

    bufs = (buf0, buf1, buf2, buf3)
    gathers = [None] * CPW
    writes = [None] * CPW
    for c in range(CPW):
        if c >= NBUF:
            writes[c - NBUF].wait()
        gathers[c] = pltpu.async_copy(
            x3.at[src_scalar(c)], bufs[c % NBUF], gsem)
        if c >= NBUF - 1:
            k = c - (NBUF - 1)
            gathers[k].wait()
            writes[k] = pltpu.async_copy(
                bufs[k % NBUF], out.at[base + k], ssem)
    for k in range(CPW - (NBUF - 1), CPW):
        gathers[k].wait()
        writes[k] = pltpu.async_copy(bufs[k % NBUF], out.at[base + k], ssem)
    for k in range(CPW - NBUF, CPW):
        writes[k].wait()


@jax.jit
def _shuffle(x3, srcs):
    run = pl.kernel(
        _shuffle_body,
        out_type=jax.ShapeDtypeStruct((NCHUNK, CH, W), jnp.float32),
        mesh=plsc.VectorSubcoreMesh(core_axis_name="c", subcore_axis_name="s"),
        scratch_types=[
            pltpu.VMEM((IDROWS, LANE), jnp.int32),
            pltpu.VMEM((CH, W), jnp.float32),
            pltpu.VMEM((CH, W), jnp.float32),
            pltpu.VMEM((CH, W), jnp.float32),
            pltpu.VMEM((CH, W), jnp.float32),
            pltpu.SemaphoreType.DMA,
            pltpu.SemaphoreType.DMA,
        ],
    )
    return run(x3, srcs)


def kernel(x, forward_shuffle_idx):
    src_plane = (jnp.arange(B, dtype=jnp.int32)[:, None] * C
                 + forward_shuffle_idx[None, :])
    src_chunk = (src_plane[:, :, None] * SPLIT
                 + jnp.arange(SPLIT, dtype=jnp.int32)).reshape(NW, CPW)
    srcs = jnp.pad(src_chunk, ((0, 0), (0, IDROWS * LANE - CPW)))
    srcs = srcs.reshape(NW, IDROWS, LANE)
    out = _shuffle(x.reshape(NCHUNK, CH, W), srcs)
    return (out.reshape(B, C, H, W), 0)


# speedup vs baseline: 33.0636x; 1.0131x over previous
import functools

import jax
import jax.numpy as jnp
from jax import lax
from jax.experimental import pallas as pl
from jax.experimental.pallas import tpu as pltpu
from jax.experimental.pallas import tpu_sc as plsc

NC = 2
NS = 16
NW = NC * NS

B, C, H, W = 4, 96, 224, 224
NPLANES = B * C
PPW = NPLANES // NW
LANE = 16


def _shuffle_body(x3, srcs, out, idx_v, buf0, buf1, gsem, ssem):
    wid = lax.axis_index("s") * NC + lax.axis_index("c")
    base = wid * PPW
    pltpu.sync_copy(srcs.at[wid], idx_v)
    ids = idx_v[...]

    def src_scalar(j):
        return ids[j]

    bufs = (buf0, buf1)
    gathers = [None] * PPW
    writes = [None] * PPW
    for j in range(PPW):
        if j >= 2:
            writes[j - 2].wait()
        gathers[j] = pltpu.async_copy(x3.at[src_scalar(j)], bufs[j % 2], gsem)
        if j >= 1:
            gathers[j - 1].wait()
            writes[j - 1] = pltpu.async_copy(
                bufs[(j - 1) % 2], out.at[base + j - 1], ssem)
    gathers[PPW - 1].wait()
    writes[PPW - 1] = pltpu.async_copy(
        bufs[(PPW - 1) % 2], out.at[base + PPW - 1], ssem)
    writes[PPW - 2].wait()
    writes[PPW - 1].wait()


@jax.jit
def _shuffle(x3, srcs):
    run = pl.kernel(
        _shuffle_body,
        out_type=jax.ShapeDtypeStruct((NPLANES, H, W), jnp.float32),
        mesh=plsc.VectorSubcoreMesh(core_axis_name="c", subcore_axis_name="s"),
        scratch_types=[
            pltpu.VMEM((LANE,), jnp.int32),
            pltpu.VMEM((H, W), jnp.float32),
            pltpu.VMEM((H, W), jnp.float32),
            pltpu.SemaphoreType.DMA,
            pltpu.SemaphoreType.DMA,
        ],
    )
    return run(x3, srcs)


def kernel(x, forward_shuffle_idx):
    src_plane = (jnp.arange(B, dtype=jnp.int32)[:, None] * C
                 + forward_shuffle_idx[None, :]).reshape(NW, PPW)
    srcs = jnp.pad(src_plane, ((0, 0), (0, LANE - PPW)))
    out = _shuffle(x.reshape(NPLANES, H, W), srcs)
    return (out.reshape(B, C, H, W), 0)


# speedup vs baseline: 33.1211x; 1.0017x over previous
import functools

import jax
import jax.numpy as jnp
from jax import lax
from jax.experimental import pallas as pl
from jax.experimental.pallas import tpu as pltpu
from jax.experimental.pallas import tpu_sc as plsc

NC = 2
NS = 16
NW = NC * NS

B, C, H, W = 4, 96, 224, 224
NPLANES = B * C
PPW = NPLANES // NW
LANE = 16


def _shuffle_body(x3, srcs, out, idx_v, buf0, buf1, gsem, ssem):
    wid = lax.axis_index("s") * NC + lax.axis_index("c")
    base = wid * PPW
    pltpu.sync_copy(srcs.at[wid], idx_v)
    ids = idx_v[...]

    def src_scalar(j):
        return ids[j]

    bufs = (buf0, buf1)
    gathers = [None] * PPW
    writes = [None] * PPW
    for j in range(PPW):
        if j >= 2:
            writes[j - 2].wait()
        gathers[j] = pltpu.async_copy(x3.at[src_scalar(j)], bufs[j % 2], gsem)
        if j >= 1:
            gathers[j - 1].wait()
            writes[j - 1] = pltpu.async_copy(
                bufs[(j - 1) % 2], out.at[base + j - 1], ssem)
    gathers[PPW - 1].wait()
    writes[PPW - 1] = pltpu.async_copy(
        bufs[(PPW - 1) % 2], out.at[base + PPW - 1], ssem)
    writes[PPW - 2].wait()
    writes[PPW - 1].wait()


@jax.jit
def _shuffle(x3, srcs):
    run = pl.kernel(
        _shuffle_body,
        out_type=jax.ShapeDtypeStruct((NPLANES, H, W), jnp.float32),
        mesh=plsc.VectorSubcoreMesh(core_axis_name="c", subcore_axis_name="s"),
        compiler_params=pltpu.CompilerParams(use_tc_tiling_on_sc=True),
        scratch_types=[
            pltpu.VMEM((LANE,), jnp.int32),
            pltpu.VMEM((H, W), jnp.float32),
            pltpu.VMEM((H, W), jnp.float32),
            pltpu.SemaphoreType.DMA,
            pltpu.SemaphoreType.DMA,
        ],
    )
    return run(x3, srcs)


def kernel(x, forward_shuffle_idx):
    src_plane = (jnp.arange(B, dtype=jnp.int32)[:, None] * C
                 + forward_shuffle_idx[None, :]).reshape(NW, PPW)
    srcs = jnp.pad(src_plane, ((0, 0), (0, LANE - PPW)))
    out = _shuffle(x.reshape(NPLANES, H, W), srcs)
    return (out.reshape(B, C, H, W), 0)
